# Initial kernel scaffold; baseline (speedup 1.0000x reference)
import jax
import jax.numpy as jnp
from jax.experimental import pallas as pl


def kernel(input_ids, additional_token_ids, base_table, add_table):
    raise NotImplementedError("write your pallas kernel here")


# speedup vs baseline: 6.0704x; 6.0704x over previous
import functools

import jax
import jax.numpy as jnp
from jax import lax
from jax.experimental import pallas as pl
from jax.experimental.pallas import tpu as pltpu
from jax.experimental.pallas import tpu_sc as plsc

_BATCH, _SEQ = 4096, 200
_VOCAB, _ADD_VOCAB, _DIM = 100000, 1024, 64
_B = _BATCH * _SEQ
_NC, _NS, _L = 2, 16, 16
_NW = _NC * _NS
_BPW = _B // _NW
_CHUNK = 512
_NCHUNK = _BPW // _CHUNK
_IDXW = 128
_NGATH = _CHUNK // _IDXW
_GPR = _IDXW // _L

_mesh = plsc.VectorSubcoreMesh(
    core_axis_name="c", subcore_axis_name="s", num_cores=_NC, num_subcores=_NS
)


@functools.partial(
    pl.kernel,
    out_type=jax.ShapeDtypeStruct((_B, _DIM), jnp.float32),
    mesh=_mesh,
    compiler_params=pltpu.CompilerParams(use_tc_tiling_on_sc=False),
    scratch_types=[
        pltpu.VMEM((_CHUNK,), jnp.int32),
        pltpu.VMEM((_CHUNK,), jnp.int32),
        pltpu.VMEM((_NGATH, _IDXW), jnp.int32),
        pltpu.VMEM((_CHUNK, _DIM), jnp.float32),
        pltpu.SemaphoreType.DMA,
        pltpu.SemaphoreType.DMA,
        pltpu.SemaphoreType.DMA,
    ],
)
def _sc_lookup(iid_hbm, aid_hbm, table_hbm, out_hbm,
               iid_v, aid_v, idx_v, rows_v, sem_i, sem_a, sem_g):
    wid = lax.axis_index("s") * _NC + lax.axis_index("c")
    base = wid * _BPW

    def chunk_body(c, carry):
        tok0 = base + c * _CHUNK
        cp_i = pltpu.async_copy(iid_hbm.at[pl.ds(tok0, _CHUNK)], iid_v, sem_i)
        cp_a = pltpu.async_copy(aid_hbm.at[pl.ds(tok0, _CHUNK)], aid_v, sem_a)
        cp_i.wait()
        cp_a.wait()
        for j in range(_NGATH):
            for h in range(_GPR):
                g = j * _GPR + h
                a = aid_v[pl.ds(g * _L, _L)]
                i = iid_v[pl.ds(g * _L, _L)]
                idx = jnp.where(a == -1, i, _VOCAB + jnp.maximum(a, 0))
                idx_v[j, pl.ds(h * _L, _L)] = idx
        gathers = [
            pltpu.async_copy(
                table_hbm.at[idx_v.at[j]],
                rows_v.at[pl.ds(j * _IDXW, _IDXW)],
                sem_g,
            )
            for j in range(_NGATH)
        ]
        for cp in gathers:
            cp.wait()
        pltpu.sync_copy(rows_v, out_hbm.at[pl.ds(tok0, _CHUNK)])
        return carry

    lax.fori_loop(0, _NCHUNK, chunk_body, 0)


def kernel(input_ids, additional_token_ids, base_table, add_table):
    combined = jnp.concatenate(
        [base_table, add_table.astype(base_table.dtype)], axis=0
    )
    iid = input_ids.reshape(_B).astype(jnp.int32)
    aid = additional_token_ids.reshape(_B).astype(jnp.int32)
    out = _sc_lookup(iid, aid, combined)
    return out.reshape(_BATCH, _SEQ, _DIM)
